# Initial kernel scaffold; baseline (speedup 1.0000x reference)
import jax
import jax.numpy as jnp
from jax.experimental import pallas as pl


def kernel(x, Wg, bg, W1, b1, W2, b2):
    raise NotImplementedError("write your pallas kernel here")


# speedup vs baseline: 1.8778x; 1.8778x over previous
import functools

import jax
import jax.numpy as jnp
from jax import lax
from jax.experimental import pallas as pl
from jax.experimental.pallas import tpu as pltpu
from jax.experimental.pallas import tpu_sc as plsc

D = 768
E = 16
F = 2048
T = 2048
M = 128
NT = T // M
NI = NT + E - 1
EP = 128

NC = 2
NS = 16
NW = NC * NS
RPW = T // NW


def _gate_body(x_ref, wg_ref, bg_ref, pos_ref, cnt_ref):
    xx = x_ref[...]
    logits = jnp.dot(xx, wg_ref[...],
                     preferred_element_type=jnp.float32) + bg_ref[...]
    lane = lax.broadcasted_iota(jnp.int32, (T, EP), 1)
    mx = jnp.max(logits, axis=-1, keepdims=True)
    e_t = jnp.min(jnp.where(logits == mx, lane, EP), axis=-1, keepdims=True)
    oh = (lane == e_t).astype(jnp.float32)

    counts = jnp.sum(oh, axis=0, keepdims=True)
    cnt_ref[...] = counts

    r = lax.broadcasted_iota(jnp.int32, (EP, EP), 0)
    c = lax.broadcasted_iota(jnp.int32, (EP, EP), 1)
    lt = (r < c).astype(jnp.float32)
    off = jnp.dot(counts, lt, preferred_element_type=jnp.float32)

    ri = lax.broadcasted_iota(jnp.int32, (M, M), 0)
    ci = lax.broadcasted_iota(jnp.int32, (M, M), 1)
    tri = (ci <= ri).astype(jnp.float32)
    base = jnp.zeros((1, EP), jnp.float32)
    for b in range(NT):
        ohb = oh[b * M:(b + 1) * M]
        incb = jnp.dot(tri, ohb, preferred_element_type=jnp.float32) + base
        rank = jnp.sum(incb * ohb, axis=-1, keepdims=True) - 1.0
        offt = jnp.sum(off * ohb, axis=-1, keepdims=True)
        pos_ref[b * M:(b + 1) * M, :] = (offt + rank).astype(jnp.int32)
        base = base + jnp.sum(ohb, axis=0, keepdims=True)


def _gate(x_flat, Wg, bg):
    wg_p = jnp.zeros((D, EP), jnp.float32).at[:, :E].set(Wg)
    bg_p = jnp.full((1, EP), -1e30, jnp.float32).at[0, :E].set(bg)
    pos2d, cnt = pl.pallas_call(
        _gate_body,
        out_shape=[
            jax.ShapeDtypeStruct((T, 1), jnp.int32),
            jax.ShapeDtypeStruct((1, EP), jnp.float32),
        ],
    )(x_flat, wg_p, bg_p)
    return pos2d.reshape(T), cnt[0, :E].astype(jnp.int32)


def _sc_kernel(body):
    return functools.partial(
        pl.kernel, body,
        mesh=plsc.VectorSubcoreMesh(core_axis_name="c",
                                    subcore_axis_name="s"),
        out_type=jax.ShapeDtypeStruct((T, D), jnp.float32),
        scratch_types=[
            pltpu.VMEM((RPW,), jnp.int32),
            pltpu.VMEM((RPW, D), jnp.float32),
            pltpu.SemaphoreType.DMA,
        ],
    )()


def _dispatch(x_flat, pos):
    def body(x_hbm, pos_hbm, xs_hbm, idx_v, rows_v, sem):
        wid = lax.axis_index("s") * NC + lax.axis_index("c")
        base = wid * RPW
        pltpu.sync_copy(pos_hbm.at[pl.ds(base, RPW)], idx_v)
        pltpu.sync_copy(x_hbm.at[pl.ds(base, RPW)], rows_v)
        pltpu.async_copy(rows_v, xs_hbm.at[idx_v], sem).wait()

    return _sc_kernel(body)(x_flat, pos)


def _combine(ys, pos):
    def body(ys_hbm, pos_hbm, out_hbm, idx_v, rows_v, sem):
        wid = lax.axis_index("s") * NC + lax.axis_index("c")
        base = wid * RPW
        pltpu.sync_copy(pos_hbm.at[pl.ds(base, RPW)], idx_v)
        pltpu.async_copy(ys_hbm.at[idx_v], rows_v, sem).wait()
        pltpu.sync_copy(rows_v, out_hbm.at[pl.ds(base, RPW)])

    return _sc_kernel(body)(ys, pos)


def _ffn_body(gI, mI, loI, hiI, fI, xs_ref, w1_ref, b1_ref, w2_ref, b2_ref,
              ys_ref):
    i = pl.program_id(0)
    h = jnp.maximum(
        jnp.dot(xs_ref[...], w1_ref[0],
                preferred_element_type=jnp.float32) + b1_ref[0], 0.0)
    y = jnp.dot(h, w2_ref[0], preferred_element_type=jnp.float32) + b2_ref[0]
    gidx = mI[i] * M + lax.broadcasted_iota(jnp.int32, (M, 1), 0)
    msk = (gidx >= loI[i]) & (gidx < hiI[i])
    contrib = jnp.where(msk, y, 0.0)

    @pl.when(fI[i] == 1)
    def _zero():
        ys_ref[...] = jnp.zeros_like(ys_ref)

    ys_ref[...] += contrib


def _ffn(meta, xs, W1, b1, W2, b2):
    gI, mI, loI, hiI, fI = meta
    grid_spec = pltpu.PrefetchScalarGridSpec(
        num_scalar_prefetch=5,
        grid=(NI,),
        in_specs=[
            pl.BlockSpec((M, D), lambda i, g, m, lo, hi, f: (m[i], 0)),
            pl.BlockSpec((1, D, F), lambda i, g, m, lo, hi, f: (g[i], 0, 0)),
            pl.BlockSpec((1, 1, F), lambda i, g, m, lo, hi, f: (g[i], 0, 0)),
            pl.BlockSpec((1, F, D), lambda i, g, m, lo, hi, f: (g[i], 0, 0)),
            pl.BlockSpec((1, 1, D), lambda i, g, m, lo, hi, f: (g[i], 0, 0)),
        ],
        out_specs=pl.BlockSpec((M, D), lambda i, g, m, lo, hi, f: (m[i], 0)),
    )
    return pl.pallas_call(
        _ffn_body,
        grid_spec=grid_spec,
        out_shape=jax.ShapeDtypeStruct((T, D), jnp.float32),
        compiler_params=pltpu.CompilerParams(
            dimension_semantics=("arbitrary",)),
    )(gI, mI, loI, hiI, fI, xs, W1, b1.reshape(E, 1, F), W2,
      b2.reshape(E, 1, D))


def _metadata(counts):
    off = jnp.cumsum(counts) - counts
    hi = off + counts
    t0 = off // M
    t1 = (hi - 1) // M
    ntiles = jnp.where(counts > 0, t1 - t0 + 1, 0)
    start = jnp.cumsum(ntiles) - ntiles
    total = jnp.sum(ntiles)
    i = jnp.arange(NI, dtype=jnp.int32)
    valid = i < total
    ic = jnp.minimum(i, total - 1)
    g = jnp.searchsorted(start, ic, side="right").astype(jnp.int32) - 1
    m = t0[g] + (ic - start[g])
    lo_i = jnp.maximum(off[g], m * M)
    hi_i = jnp.minimum(hi[g], (m + 1) * M)
    lo_i = jnp.where(valid, lo_i, T)
    hi_i = jnp.where(valid, hi_i, T)
    first = jnp.concatenate(
        [jnp.ones((1,), jnp.int32), (m[1:] != m[:-1]).astype(jnp.int32)])
    return (g.astype(jnp.int32), m.astype(jnp.int32),
            lo_i.astype(jnp.int32), hi_i.astype(jnp.int32), first)


def kernel(x, Wg, bg, W1, b1, W2, b2):
    B, S, _ = x.shape
    x_flat = x.reshape(T, D)
    pos, counts = _gate(x_flat, Wg, bg)
    meta = _metadata(counts)
    xs = _dispatch(x_flat, pos)
    ys = _ffn(meta, xs, W1, b1, W2, b2)
    out = _combine(ys, pos)
    return out.reshape(B, S, D)


# speedup vs baseline: 2.3968x; 1.2764x over previous
import functools

import jax
import jax.numpy as jnp
from jax import lax
from jax.experimental import pallas as pl
from jax.experimental.pallas import tpu as pltpu
from jax.experimental.pallas import tpu_sc as plsc

D = 768
E = 16
F = 2048
T = 2048
M = 128
NT = T // M
EP = 128

NC = 2
NS = 16
NW = NC * NS
RPW = T // NW


def _gate_body(x_ref, wg_ref, bg_ref, pos_ref, off_ref, hi_ref):
    xx = x_ref[...]
    logits16 = jnp.dot(xx, wg_ref[...],
                       preferred_element_type=jnp.float32) + bg_ref[...]
    logits = jnp.concatenate(
        [logits16, jnp.full((T, EP - E), -1e30, jnp.float32)], axis=1)
    lane = lax.broadcasted_iota(jnp.int32, (T, EP), 1)
    mx = jnp.max(logits, axis=-1, keepdims=True)
    e_t = jnp.min(jnp.where(logits == mx, lane, EP), axis=-1, keepdims=True)
    oh = (lane == e_t).astype(jnp.float32)

    counts = jnp.sum(oh, axis=0, keepdims=True)

    r = lax.broadcasted_iota(jnp.int32, (EP, EP), 0)
    c = lax.broadcasted_iota(jnp.int32, (EP, EP), 1)
    lt = (r < c).astype(jnp.float32)
    off = jnp.dot(counts, lt, preferred_element_type=jnp.float32)
    off_ref[...] = off.astype(jnp.int32)
    hi_ref[...] = (off + counts).astype(jnp.int32)

    ri = lax.broadcasted_iota(jnp.int32, (M, M), 0)
    ci = lax.broadcasted_iota(jnp.int32, (M, M), 1)
    tri = (ci <= ri).astype(jnp.float32)
    base = jnp.zeros((1, EP), jnp.float32)
    for b in range(NT):
        ohb = oh[b * M:(b + 1) * M]
        incb = jnp.dot(tri, ohb, preferred_element_type=jnp.float32) + base
        rank = jnp.sum(incb * ohb, axis=-1, keepdims=True) - 1.0
        offt = jnp.sum(off * ohb, axis=-1, keepdims=True)
        pos_ref[b * M:(b + 1) * M, :] = (offt + rank).astype(jnp.int32)
        base = base + jnp.sum(ohb, axis=0, keepdims=True)


def _gate(x_flat, Wg, bg):
    pos2d, off, hi = pl.pallas_call(
        _gate_body,
        out_shape=[
            jax.ShapeDtypeStruct((T, 1), jnp.int32),
            jax.ShapeDtypeStruct((1, EP), jnp.int32),
            jax.ShapeDtypeStruct((1, EP), jnp.int32),
        ],
    )(x_flat, Wg, bg.reshape(1, E))
    return pos2d.reshape(T), off, hi


def _sc_kernel(body):
    return functools.partial(
        pl.kernel, body,
        mesh=plsc.VectorSubcoreMesh(core_axis_name="c",
                                    subcore_axis_name="s"),
        out_type=jax.ShapeDtypeStruct((T, D), jnp.float32),
        scratch_types=[
            pltpu.VMEM((RPW,), jnp.int32),
            pltpu.VMEM((RPW, D), jnp.float32),
            pltpu.SemaphoreType.DMA,
        ],
    )()


def _dispatch(x_flat, pos):
    def body(x_hbm, pos_hbm, xs_hbm, idx_v, rows_v, sem):
        wid = lax.axis_index("s") * NC + lax.axis_index("c")
        base = wid * RPW
        pltpu.sync_copy(pos_hbm.at[pl.ds(base, RPW)], idx_v)
        pltpu.sync_copy(x_hbm.at[pl.ds(base, RPW)], rows_v)
        pltpu.async_copy(rows_v, xs_hbm.at[idx_v], sem).wait()

    return _sc_kernel(body)(x_flat, pos)


def _combine(ys, pos):
    def body(ys_hbm, pos_hbm, out_hbm, idx_v, rows_v, sem):
        wid = lax.axis_index("s") * NC + lax.axis_index("c")
        base = wid * RPW
        pltpu.sync_copy(pos_hbm.at[pl.ds(base, RPW)], idx_v)
        pltpu.async_copy(ys_hbm.at[idx_v], rows_v, sem).wait()
        pltpu.sync_copy(rows_v, out_hbm.at[pl.ds(base, RPW)])

    return _sc_kernel(body)(ys, pos)


def _ffn_body(off_ref, hi_ref, xs_ref, w1_ref, b1_ref, w2_ref, b2_ref,
              ys_ref):
    g = pl.program_id(0)

    @pl.when(g == 0)
    def _zero():
        ys_ref[...] = jnp.zeros_like(ys_ref)

    lo = off_ref[0, g]
    hi = hi_ref[0, g]
    t0 = lo // M
    t1 = lax.select(hi > lo, (hi + M - 1) // M, t0)

    def tile_step(t, _):
        row = t * M
        xt = xs_ref[pl.ds(row, M), :]
        h = jnp.maximum(
            jnp.dot(xt, w1_ref[0], preferred_element_type=jnp.float32)
            + b1_ref[0], 0.0)
        y = (jnp.dot(h, w2_ref[0], preferred_element_type=jnp.float32)
             + b2_ref[0])
        gidx = row + lax.broadcasted_iota(jnp.int32, (M, 1), 0)
        msk = (gidx >= lo) & (gidx < hi)
        ys_ref[pl.ds(row, M), :] += jnp.where(msk, y, 0.0)
        return 0

    lax.fori_loop(t0, t1, tile_step, 0)


def _ffn(off, hi, xs, W1, b1, W2, b2):
    grid_spec = pltpu.PrefetchScalarGridSpec(
        num_scalar_prefetch=2,
        grid=(E,),
        in_specs=[
            pl.BlockSpec((T, D), lambda g, o, h: (0, 0)),
            pl.BlockSpec((1, D, F), lambda g, o, h: (g, 0, 0)),
            pl.BlockSpec((1, 1, F), lambda g, o, h: (g, 0, 0)),
            pl.BlockSpec((1, F, D), lambda g, o, h: (g, 0, 0)),
            pl.BlockSpec((1, 1, D), lambda g, o, h: (g, 0, 0)),
        ],
        out_specs=pl.BlockSpec((T, D), lambda g, o, h: (0, 0)),
    )
    return pl.pallas_call(
        _ffn_body,
        grid_spec=grid_spec,
        out_shape=jax.ShapeDtypeStruct((T, D), jnp.float32),
        compiler_params=pltpu.CompilerParams(
            dimension_semantics=("arbitrary",)),
    )(off, hi, xs, W1, b1.reshape(E, 1, F), W2, b2.reshape(E, 1, D))


def kernel(x, Wg, bg, W1, b1, W2, b2):
    B, S, _ = x.shape
    x_flat = x.reshape(T, D)
    pos, off, hi = _gate(x_flat, Wg, bg)
    xs = _dispatch(x_flat, pos)
    ys = _ffn(off, hi, xs, W1, b1, W2, b2)
    out = _combine(ys, pos)
    return out.reshape(B, S, D)
